# baseline (speedup 1.0000x reference)
import functools
import jax
import jax.numpy as jnp
import numpy as np
from jax import lax
from jax.experimental import pallas as pl
from jax.experimental.pallas import tpu as pltpu
from jax.experimental.pallas import tpu_sc as plsc

_P = 2
_B = 64
_A = 32
_N = _B * _A
_NBT = 64
_NAT = 128
_TI = 256
_NT = _N // _TI
_TB = _TI // _A
_C_LK = 1.0 / (2.0 * np.pi ** 1.5)

_NW = 32
_BPW = (_P * _B) // _NW
_APW = _BPW * _A


def _sc_prep_body(tp_hbm, coords_hbm, bt_hbm, bta_hbm, btpd_hbm,
                  feats_hbm, cpf_hbm, wtrue_hbm,
                  tp_v, cv, btv, bta_v, pdv, featsbuf, cpbuf, wbuf, sem):
    wid = lax.axis_index("c") * 16 + lax.axis_index("s")
    p = wid // 16
    b0 = (wid % 16) * _BPW
    a0 = b0 * _A

    pltpu.sync_copy(tp_hbm, tp_v)
    pltpu.sync_copy(bt_hbm, btv)
    pltpu.sync_copy(bta_hbm, bta_v)
    pltpu.sync_copy(coords_hbm.at[p, pl.ds(a0 * 3, _APW * 3)], cv)

    iota = jnp.arange(16, dtype=jnp.int32)
    zeros = jnp.zeros((16,), jnp.float32)

    for j in range(_BPW):
        fb16 = jnp.full((16,), p * _B + b0 + j, jnp.int32)
        bt16 = plsc.load_gather(btv, [fb16])
        bt_s = jnp.max(bt16)

        pltpu.sync_copy(btpd_hbm.at[bt_s], pdv)
        for r in range(_A):
            for h in range(2):
                pd16 = pdv[pl.ds(r * _A + h * 16, 16)]
                w16 = jnp.where(pd16 > 4, 1.0,
                                jnp.where(pd16 == 4, 0.2, 0.0))
                flat = r * _A + h * 16 + iota
                w16 = jnp.where(flat % (_A + 1) == 0, 0.0, w16)
                wbuf[pl.ds(r * _A + h * 16, 16)] = w16
        pltpu.sync_copy(wbuf, wtrue_hbm.at[p, pl.ds((b0 + j) * _A * _A, _A * _A)])

        for h in range(2):
            aw16 = jnp.full((16,), j * _A + h * 16, jnp.int32) + iota
            at16 = plsc.load_gather(bta_v, [bt16 * _A + h * 16 + iota])
            x = plsc.load_gather(cv, [aw16 * 3])
            y = plsc.load_gather(cv, [aw16 * 3 + 1])
            z = plsc.load_gather(cv, [aw16 * 3 + 2])
            n2 = x * x + y * y + z * z
            cols = [x, y, z, zeros, zeros, zeros, zeros, zeros, n2]
            for c in range(9, 14):
                cols.append(plsc.load_gather(tp_v, [at16 * 16 + c]))
            cols.append(zeros)
            cols.append(zeros)
            for c in range(16):
                plsc.store_scatter(featsbuf, [aw16 * 16 + c], cols[c])
                plsc.store_scatter(cpbuf, [aw16 + c * _APW], cols[c])

    pltpu.sync_copy(featsbuf, feats_hbm.at[p, pl.ds(a0 * 16, _APW * 16)])
    copies = [pltpu.async_copy(cpbuf.at[pl.ds(c * _APW, _APW)],
                               cpf_hbm.at[p, pl.ds(c * _N + a0, _APW)], sem)
              for c in range(16)]
    for cp_ in copies:
        cp_.wait()


def _sc_prep(tp16, coords, block_types, bt_atom_types, bt_path_distance):
    mesh = plsc.VectorSubcoreMesh(core_axis_name="c", subcore_axis_name="s",
                                  num_cores=2, num_subcores=16)
    feats, cpf, wtrue = pl.kernel(
        _sc_prep_body,
        out_type=(
            jax.ShapeDtypeStruct((_P, _N * 16), jnp.float32),
            jax.ShapeDtypeStruct((_P, 16 * _N), jnp.float32),
            jax.ShapeDtypeStruct((_P, _N * _A), jnp.float32),
        ),
        mesh=mesh,
        compiler_params=pltpu.CompilerParams(needs_layout_passes=False),
        scratch_types=[
            pltpu.VMEM((_NAT * 16,), jnp.float32),
            pltpu.VMEM((_APW * 3,), jnp.float32),
            pltpu.VMEM((_P * _B,), jnp.int32),
            pltpu.VMEM((_NBT * _A,), jnp.int32),
            pltpu.VMEM((_A * _A,), jnp.int32),
            pltpu.VMEM((_APW * 16,), jnp.float32),
            pltpu.VMEM((16 * _APW,), jnp.float32),
            pltpu.VMEM((_A * _A,), jnp.float32),
            pltpu.SemaphoreType.DMA,
        ],
    )(tp16.reshape(-1), coords.reshape(_P, _N * 3), block_types.reshape(-1),
      bt_atom_types.reshape(-1), bt_path_distance.reshape(_NBT, _A * _A))
    return (feats.reshape(_P, _N, 16), cpf.reshape(_P, 16, _N),
            wtrue.reshape(_P, _N, _A))


def _pair_tile_kernel(rp_ref, cp_ref, mbb_ref, wt_ref, out_ref):
    t = pl.program_id(1)
    rp = rp_ref[0]
    cp = cp_ref[0]
    mb = mbb_ref[0]
    dl = wt_ref[0]

    cross = jnp.dot(rp[:, 0:8], cp[0:8, :], preferred_element_type=jnp.float32,
                    precision=jax.lax.Precision.HIGHEST)
    n2i = rp[:, 8:9]
    n2j = cp[8:9, :]
    t2 = jnp.maximum(n2i + n2j - 2.0 * cross, 0.0) + 1e-8
    rs = jax.lax.rsqrt(t2)
    d = jnp.maximum(t2 * rs, 0.8)
    inv_d = jnp.minimum(rs, 1.25)
    inv_d2 = inv_d * inv_d

    ri = rp[:, 9:10]
    rj = cp[9:10, :]
    sd = (ri + rj) * inv_d
    sd2 = sd * sd
    sd6 = sd2 * sd2 * sd2
    lj = (rp[:, 10:11] * cp[10:11, :]) * (sd6 * sd6 - 2.0 * sd6)

    xi = (d - ri) * rp[:, 11:12]
    xj = (d - rj) * cp[11:12, :]
    ei = jnp.exp(-xi * xi)
    ej = jnp.exp(-xj * xj)
    lk = inv_d2 * (rp[:, 12:13] * cp[13:14, :] * ei
                   + cp[12:13, :] * rp[:, 13:14] * ej)

    wt = jnp.where(mb > 4, 1.0, jnp.where(mb == 4, 0.2, 0.0))
    rowi = jax.lax.broadcasted_iota(jnp.int32, (_TI, 1), 0)
    colj = jax.lax.broadcasted_iota(jnp.int32, (1, _N), 1)
    same = (rowi // _A + t * _TB) == (colj // _A)
    er_i = jax.lax.broadcasted_iota(jnp.int32, (_TI, _TB), 0) // _A
    er_j = jax.lax.broadcasted_iota(jnp.int32, (_TI, _TB), 1)
    erow = (er_i == er_j).astype(jnp.float32)
    cc_i = jax.lax.broadcasted_iota(jnp.int32, (_B, _N), 0)
    cc_j = jax.lax.broadcasted_iota(jnp.int32, (_B, _N), 1) // _A
    ccol = (cc_i == cc_j).astype(jnp.float32)
    w_exp = jnp.dot(jnp.dot(erow, wt, preferred_element_type=jnp.float32),
                    ccol, preferred_element_type=jnp.float32)
    tm_i = jax.lax.broadcasted_iota(jnp.int32, (_A, _N), 0)
    tm_j = jax.lax.broadcasted_iota(jnp.int32, (_A, _N), 1) % _A
    tmat = (tm_i == tm_j).astype(jnp.float32)
    d_exp = jnp.dot(dl, tmat, preferred_element_type=jnp.float32)
    w = jnp.where(same, d_exp, w_exp)

    wm = jnp.where(d < 6.0, w, 0.0)
    lj_s = jnp.sum(lj * wm)
    lk_s = jnp.sum(lk * wm)

    ii = jax.lax.broadcasted_iota(jnp.int32, (8, 128), 0)
    upd = jnp.where(ii == 0, lj_s, 0.0) + jnp.where(ii == 1, lk_s, 0.0)

    @pl.when(t == 0)
    def _init():
        out_ref[0] = upd

    @pl.when(t != 0)
    def _acc():
        out_ref[0] = out_ref[0] + upd


def _pairwise_call(rp, cpf, mbb, wtrue, interpret=False):
    return pl.pallas_call(
        _pair_tile_kernel,
        grid=(_P, _NT),
        in_specs=[
            pl.BlockSpec((1, _TI, 16), lambda p, t: (p, t, 0)),
            pl.BlockSpec((1, 16, _N), lambda p, t: (p, 0, 0)),
            pl.BlockSpec((1, _TB, _B), lambda p, t: (p, t, 0)),
            pl.BlockSpec((1, _TI, _A), lambda p, t: (p, t, 0)),
        ],
        out_specs=pl.BlockSpec((1, 8, 128), lambda p, t: (p, 0, 0)),
        out_shape=jax.ShapeDtypeStruct((_P, 8, 128), jnp.float32),
        interpret=interpret,
    )(rp, cpf, mbb, wtrue)


def kernel(coords, type_params, global_params, block_types, min_block_bondsep,
           bt_atom_types, bt_path_distance):
    r = type_params[:, 0]
    sw = jnp.sqrt(type_params[:, 1])
    invlam = 1.0 / type_params[:, 3]
    h = (type_params[:, 7] < 0.5).astype(jnp.float32)
    a = _C_LK * h * type_params[:, 2] * invlam
    vol = h * type_params[:, 4]
    zc = jnp.zeros((_NAT,), jnp.float32)
    tp16 = jnp.stack([zc, zc, zc, zc, zc, zc, zc, zc, zc,
                      r, sw, invlam, a, vol, zc, zc], axis=1)

    feats, cpf, wtrue = _sc_prep(tp16, coords,
                                 block_types.astype(jnp.int32),
                                 bt_atom_types.astype(jnp.int32),
                                 bt_path_distance.astype(jnp.int32))

    out = _pairwise_call(feats, cpf, min_block_bondsep.astype(jnp.int32), wtrue)
    return jnp.stack([0.5 * out[:, 0, 0], 0.5 * out[:, 1, 0]], axis=0)
